# baseline (speedup 1.0000x reference)
import jax
import jax.numpy as jnp
from jax.experimental import pallas as pl
from jax.experimental.pallas import tpu as pltpu
from jax.experimental.pallas import tpu_sc as plsc

BATCH = 16384
FEAT = 64
WINDOW = 128


def kernel(item_index, mu_table, logvar_table):
    idx = item_index.reshape(1, BATCH).astype(jnp.int32)
    mesh = plsc.VectorSubcoreMesh(core_axis_name="core", subcore_axis_name="subcore")
    out_type = (
        jax.ShapeDtypeStruct((BATCH, FEAT), mu_table.dtype),
        jax.ShapeDtypeStruct((BATCH, FEAT), logvar_table.dtype),
    )

    @pl.kernel(
        out_type=out_type,
        mesh=mesh,
        compiler_params=pltpu.CompilerParams(use_tc_tiling_on_sc=False),
    )
    def sc_gather(mu_hbm, lv_hbm, i_hbm, mu_out, lv_out):
        def body(i_vmem, mu_vmem, lv_vmem):
            pltpu.sync_copy(mu_hbm.at[i_vmem.at[0]], mu_vmem)
            pltpu.sync_copy(lv_hbm.at[i_vmem.at[0]], lv_vmem)

        pltpu.emit_pipeline(
            body,
            grid=(BATCH // WINDOW,),
            in_specs=[pl.BlockSpec((1, WINDOW), lambda i: (0, i))],
            out_specs=[
                pl.BlockSpec((WINDOW, FEAT), lambda i: (i, 0)),
                pl.BlockSpec((WINDOW, FEAT), lambda i: (i, 0)),
            ],
            core_axis_name=("core", "subcore"),
            dimension_semantics=(pltpu.PARALLEL,),
        )(i_hbm, mu_out, lv_out)

    return sc_gather(mu_table, logvar_table, idx)


# speedup vs baseline: 1.0053x; 1.0053x over previous
import jax
import jax.numpy as jnp
from jax import lax
from jax.experimental import pallas as pl
from jax.experimental.pallas import tpu as pltpu
from jax.experimental.pallas import tpu_sc as plsc

BATCH = 16384
FEAT = 64
NC, NS = 2, 16
NW = NC * NS
B_PER_W = BATCH // NW
CH = 128
NCH = B_PER_W // CH


def kernel(item_index, mu_table, logvar_table):
    idx = item_index.reshape(NW * NCH, CH).astype(jnp.int32)
    mesh = plsc.VectorSubcoreMesh(core_axis_name="core", subcore_axis_name="subcore")
    out_type = (
        jax.ShapeDtypeStruct((BATCH, FEAT), mu_table.dtype),
        jax.ShapeDtypeStruct((BATCH, FEAT), logvar_table.dtype),
    )

    @pl.kernel(
        out_type=out_type,
        mesh=mesh,
        scratch_types=[
            pltpu.VMEM((NCH, CH), jnp.int32),
            pltpu.VMEM((B_PER_W, FEAT), jnp.float32),
            pltpu.VMEM((B_PER_W, FEAT), jnp.float32),
            pltpu.SemaphoreType.DMA,
            pltpu.SemaphoreType.DMA,
        ],
        compiler_params=pltpu.CompilerParams(use_tc_tiling_on_sc=False),
    )
    def sc_gather(mu_hbm, lv_hbm, i_hbm, mu_out, lv_out, idx_v, mu_v, lv_v, sem, osem):
        wid = lax.axis_index("subcore") * NC + lax.axis_index("core")
        base = wid * B_PER_W
        pltpu.sync_copy(i_hbm.at[pl.ds(wid * NCH, NCH)], idx_v)
        copies = []
        for j in range(NCH):
            rows = pl.ds(j * CH, CH)
            copies.append(pltpu.async_copy(mu_hbm.at[idx_v.at[j]], mu_v.at[rows], sem))
            copies.append(pltpu.async_copy(lv_hbm.at[idx_v.at[j]], lv_v.at[rows], sem))
        out_copies = []
        for j in range(NCH):
            copies[2 * j].wait()
            copies[2 * j + 1].wait()
            rows = pl.ds(j * CH, CH)
            orows = pl.ds(base + j * CH, CH)
            out_copies.append(pltpu.async_copy(mu_v.at[rows], mu_out.at[orows], osem))
            out_copies.append(pltpu.async_copy(lv_v.at[rows], lv_out.at[orows], osem))
        for c in out_copies:
            c.wait()

    return sc_gather(mu_table, logvar_table, idx)


# speedup vs baseline: 1.2187x; 1.2122x over previous
import jax
import jax.numpy as jnp
from jax import lax
from jax.experimental import pallas as pl
from jax.experimental.pallas import tpu as pltpu
from jax.experimental.pallas import tpu_sc as plsc

BATCH = 16384
FEAT = 64
NC, NS = 2, 16
NW = NC * NS
B_PER_W = BATCH // NW
CH = 128
NCH = B_PER_W // CH


def kernel(item_index, mu_table, logvar_table):
    idx = item_index.reshape(NW * NCH, CH).astype(jnp.int32)
    comb = jnp.concatenate([mu_table, logvar_table], axis=1)
    mesh = plsc.VectorSubcoreMesh(core_axis_name="core", subcore_axis_name="subcore")
    out_type = jax.ShapeDtypeStruct((BATCH, 2 * FEAT), jnp.float32)

    @pl.kernel(
        out_type=out_type,
        mesh=mesh,
        scratch_types=[
            pltpu.VMEM((NCH, CH), jnp.int32),
            pltpu.VMEM((B_PER_W, 2 * FEAT), jnp.float32),
            pltpu.SemaphoreType.DMA,
        ],
        compiler_params=pltpu.CompilerParams(use_tc_tiling_on_sc=True),
    )
    def sc_gather(t_hbm, i_hbm, o_hbm, idx_v, rows_v, sem):
        wid = lax.axis_index("subcore") * NC + lax.axis_index("core")
        base = wid * B_PER_W
        pltpu.sync_copy(i_hbm.at[pl.ds(wid * NCH, NCH)], idx_v)
        copies = [
            pltpu.async_copy(t_hbm.at[idx_v.at[j]], rows_v.at[pl.ds(j * CH, CH)], sem)
            for j in range(NCH)
        ]
        for c in copies:
            c.wait()
        pltpu.sync_copy(rows_v, o_hbm.at[pl.ds(base, B_PER_W)])

    out = sc_gather(comb, idx)
    return out[:, :FEAT], out[:, FEAT:]


# speedup vs baseline: 2.2940x; 1.8824x over previous
import jax
import jax.numpy as jnp
from jax import lax
from jax.experimental import pallas as pl
from jax.experimental.pallas import tpu as pltpu
from jax.experimental.pallas import tpu_sc as plsc

BATCH = 16384
FEAT = 64
NC, NS = 2, 16
NW = NC * NS
B_PER_W = BATCH // NW
PG = 128
ROWS_PER_PAGE = 128
NPAGES = B_PER_W // ROWS_PER_PAGE


def kernel(item_index, mu_table, logvar_table):
    idx = item_index.reshape(BATCH).astype(jnp.int32)
    tt_mu = mu_table.T
    tt_lv = logvar_table.T
    mesh = plsc.VectorSubcoreMesh(core_axis_name="core", subcore_axis_name="subcore")
    out_type = jax.ShapeDtypeStruct((BATCH, 2 * FEAT), jnp.float32)

    @pl.kernel(
        out_type=out_type,
        mesh=mesh,
        scratch_types=[
            pltpu.VMEM((B_PER_W,), jnp.int32),
            pltpu.VMEM((2, 2, FEAT, PG), jnp.float32),
            pltpu.VMEM((2, 2, FEAT, PG), jnp.float32),
            pltpu.VMEM((ROWS_PER_PAGE, 2 * FEAT), jnp.float32),
            pltpu.SemaphoreType.DMA,
            pltpu.SemaphoreType.DMA,
        ],
        compiler_params=pltpu.CompilerParams(
            use_tc_tiling_on_sc=True, needs_layout_passes=False
        ),
    )
    def sc_gather(mu_hbm, lv_hbm, i_hbm, o_hbm, idx_v, bufa, bufb, stage, sema, semb):
        wid = lax.axis_index("subcore") * NC + lax.axis_index("core")
        base = wid * B_PER_W
        pltpu.sync_copy(i_hbm.at[pl.ds(base, B_PER_W)], idx_v)

        iotas = [lax.iota(jnp.int32, 16) + 16 * c for c in range(4)]

        def sload(pos):
            g = (pos >> 4) << 4
            v = idx_v[pl.ds(g, 16)]
            m = lax.iota(jnp.int32, 16) == (pos - g)
            return jnp.sum(jnp.where(m, v, 0))

        def fire(buf, sem, pos):
            for k in range(2):
                i = sload(pos + k)
                po = pl.multiple_of((i >> 7) * PG, PG)
                pltpu.async_copy(mu_hbm.at[:, pl.ds(po, PG)], buf.at[k, 0], sem)
                pltpu.async_copy(lv_hbm.at[:, pl.ds(po, PG)], buf.at[k, 1], sem)

        def drain(buf, sem):
            for _ in range(4):
                pltpu.make_async_copy(mu_hbm.at[:, pl.ds(0, PG)], buf.at[0, 0], sem).wait()

        def extract(buf, pos):
            for k in range(2):
                i = sload(pos + k)
                lanev = jnp.zeros((16,), jnp.int32) + (i & 127)
                slot = (pos + k) & (ROWS_PER_PAGE - 1)
                for t in range(2):
                    for c in range(4):
                        v = plsc.load_gather(buf.at[k, t], [iotas[c], lanev])
                        stage[slot, pl.ds(t * FEAT + 16 * c, 16)] = v

        fire(bufa, sema, 0)

        @pl.loop(0, B_PER_W // 4)
        def _(m):
            p = m * 4
            fire(bufb, semb, p + 2)
            drain(bufa, sema)
            extract(bufa, p)

            @pl.when(m < B_PER_W // 4 - 1)
            def _():
                fire(bufa, sema, p + 4)

            drain(bufb, semb)
            extract(bufb, p + 2)

            @pl.when((m & 31) == 31)
            def _():
                pltpu.sync_copy(
                    stage, o_hbm.at[pl.ds(base + (m // 32) * ROWS_PER_PAGE, ROWS_PER_PAGE)]
                )

    out = sc_gather(tt_mu, tt_lv, idx)
    return out[:, :FEAT], out[:, FEAT:]


# speedup vs baseline: 2.4858x; 1.0836x over previous
import jax
import jax.numpy as jnp
from jax import lax
from jax.experimental import pallas as pl
from jax.experimental.pallas import tpu as pltpu
from jax.experimental.pallas import tpu_sc as plsc

BATCH = 16384
FEAT = 64
NC, NS = 2, 16
NW = NC * NS
B_PER_W = BATCH // NW
PG = 128
ROWS_PER_PAGE = 128
NPAGES = B_PER_W // ROWS_PER_PAGE


def kernel(item_index, mu_table, logvar_table):
    idx = item_index.reshape(BATCH).astype(jnp.int32)
    tt_mu = mu_table.T
    tt_lv = logvar_table.T
    mesh = plsc.VectorSubcoreMesh(core_axis_name="core", subcore_axis_name="subcore")
    out_type = jax.ShapeDtypeStruct((BATCH, 2 * FEAT), jnp.float32)

    @pl.kernel(
        out_type=out_type,
        mesh=mesh,
        scratch_types=[
            pltpu.VMEM((B_PER_W,), jnp.int32),
            pltpu.VMEM((2, 2, FEAT, PG), jnp.float32),
            pltpu.VMEM((2, 2, FEAT, PG), jnp.float32),
            pltpu.VMEM((2, 2, FEAT, PG), jnp.float32),
            pltpu.VMEM((ROWS_PER_PAGE, 2 * FEAT), jnp.float32),
            pltpu.SemaphoreType.DMA,
            pltpu.SemaphoreType.DMA,
            pltpu.SemaphoreType.DMA,
        ],
        compiler_params=pltpu.CompilerParams(
            use_tc_tiling_on_sc=True, needs_layout_passes=False
        ),
    )
    def sc_gather(
        mu_hbm, lv_hbm, i_hbm, o_hbm, idx_v, bufa, bufb, bufc, stage, sema, semb, semc
    ):
        wid = lax.axis_index("subcore") * NC + lax.axis_index("core")
        base = wid * B_PER_W
        pltpu.sync_copy(i_hbm.at[pl.ds(base, B_PER_W)], idx_v)

        iotas = [lax.iota(jnp.int32, 16) + 16 * c for c in range(4)]

        def sload(pos):
            g = (pos >> 4) << 4
            v = idx_v[pl.ds(g, 16)]
            m = lax.iota(jnp.int32, 16) == (pos - g)
            return jnp.sum(jnp.where(m, v, 0))

        def fire(buf, sem, pos):
            for k in range(2):
                i = sload(pos + k)
                po = pl.multiple_of((i >> 7) * PG, PG)
                pltpu.async_copy(mu_hbm.at[:, pl.ds(po, PG)], buf.at[k, 0], sem)
                pltpu.async_copy(lv_hbm.at[:, pl.ds(po, PG)], buf.at[k, 1], sem)

        def drain(buf, sem):
            for _ in range(4):
                pltpu.make_async_copy(mu_hbm.at[:, pl.ds(0, PG)], buf.at[0, 0], sem).wait()

        def extract(buf, pos):
            for k in range(2):
                i = sload(pos + k)
                lanev = jnp.zeros((16,), jnp.int32) + (i & 127)
                slot = (pos + k) & (ROWS_PER_PAGE - 1)
                for t in range(2):
                    for c in range(4):
                        v = plsc.load_gather(buf.at[k, t], [iotas[c], lanev])
                        stage[slot, pl.ds(t * FEAT + 16 * c, 16)] = v

        def maybe_flush(last_pos):
            @pl.when((last_pos & (ROWS_PER_PAGE - 1)) == ROWS_PER_PAGE - 1)
            def _():
                pg_row = base + (last_pos >> 7) * ROWS_PER_PAGE
                pltpu.sync_copy(stage, o_hbm.at[pl.ds(pg_row, ROWS_PER_PAGE)])

        fire(bufa, sema, 0)
        fire(bufb, semb, 2)
        fire(bufc, semc, 4)

        @pl.loop(0, 85)
        def _(m):
            p = m * 6
            drain(bufa, sema)
            extract(bufa, p)
            maybe_flush(p + 1)
            fire(bufa, sema, p + 6)
            drain(bufb, semb)
            extract(bufb, p + 2)
            maybe_flush(p + 3)

            @pl.when(p + 8 < B_PER_W)
            def _():
                fire(bufb, semb, p + 8)

            drain(bufc, semc)
            extract(bufc, p + 4)
            maybe_flush(p + 5)

            @pl.when(p + 10 < B_PER_W)
            def _():
                fire(bufc, semc, p + 10)

        drain(bufa, sema)
        extract(bufa, B_PER_W - 2)
        maybe_flush(B_PER_W - 1)

    out = sc_gather(tt_mu, tt_lv, idx)
    return out[:, :FEAT], out[:, FEAT:]


# speedup vs baseline: 2.8053x; 1.1286x over previous
import jax
import jax.numpy as jnp
from jax import lax
from jax.experimental import pallas as pl
from jax.experimental.pallas import tpu as pltpu
from jax.experimental.pallas import tpu_sc as plsc

BATCH = 16384
FEAT = 64
NC, NS = 2, 16
NW = NC * NS
B_PER_W = BATCH // NW
PG = 128
ROWS_PER_PAGE = 128
NPAGES = B_PER_W // ROWS_PER_PAGE


def kernel(item_index, mu_table, logvar_table):
    idx = item_index.reshape(BATCH).astype(jnp.int32)
    tt_mu = mu_table.T
    tt_lv = logvar_table.T
    mesh = plsc.VectorSubcoreMesh(core_axis_name="core", subcore_axis_name="subcore")
    out_type = jax.ShapeDtypeStruct((BATCH, 2 * FEAT), jnp.float32)

    @pl.kernel(
        out_type=out_type,
        mesh=mesh,
        scratch_types=[
            pltpu.VMEM((B_PER_W,), jnp.int32),
            pltpu.VMEM((6, 2, FEAT, PG), jnp.float32),
            pltpu.VMEM((ROWS_PER_PAGE, 2 * FEAT), jnp.float32),
            pltpu.SemaphoreType.DMA((6,)),
        ],
        compiler_params=pltpu.CompilerParams(
            use_tc_tiling_on_sc=True, needs_layout_passes=False
        ),
    )
    def sc_gather(mu_hbm, lv_hbm, i_hbm, o_hbm, idx_v, bufs, stage, sems):
        wid = lax.axis_index("subcore") * NC + lax.axis_index("core")
        base = wid * B_PER_W
        pltpu.sync_copy(i_hbm.at[pl.ds(base, B_PER_W)], idx_v)

        iotas = [lax.iota(jnp.int32, 16) + 16 * c for c in range(4)]

        def sload(pos):
            g = (pos >> 4) << 4
            v = idx_v[pl.ds(g, 16)]
            m = lax.iota(jnp.int32, 16) == (pos - g)
            return jnp.sum(jnp.where(m, v, 0))

        def fire(s, pos):
            i = sload(pos)
            po = pl.multiple_of((i >> 7) * PG, PG)
            pltpu.async_copy(mu_hbm.at[:, pl.ds(po, PG)], bufs.at[s, 0], sems.at[s])
            pltpu.async_copy(lv_hbm.at[:, pl.ds(po, PG)], bufs.at[s, 1], sems.at[s])

        def drain(s):
            for _ in range(2):
                pltpu.make_async_copy(
                    mu_hbm.at[:, pl.ds(0, PG)], bufs.at[s, 0], sems.at[s]
                ).wait()

        def extract(s, pos):
            i = sload(pos)
            lanev = jnp.zeros((16,), jnp.int32) + (i & 127)
            slot = pos & (ROWS_PER_PAGE - 1)
            for t in range(2):
                for c in range(4):
                    v = plsc.load_gather(bufs.at[s, t], [iotas[c], lanev])
                    stage[slot, pl.ds(t * FEAT + 16 * c, 16)] = v

        def maybe_flush(last_pos):
            @pl.when((last_pos & (ROWS_PER_PAGE - 1)) == ROWS_PER_PAGE - 1)
            def _():
                pg_row = base + (last_pos >> 7) * ROWS_PER_PAGE
                pltpu.sync_copy(stage, o_hbm.at[pl.ds(pg_row, ROWS_PER_PAGE)])

        for s in range(6):
            fire(s, s)

        @pl.loop(0, 84)
        def _(m):
            p = m * 6
            for s in range(6):
                drain(s)
                extract(s, p + s)
                maybe_flush(p + s)
                fire(s, p + s + 6)

        for s in range(6):
            p = 504 + s
            drain(s)
            extract(s, p)
            maybe_flush(p)

            @pl.when(p + 6 < B_PER_W)
            def _():
                fire(s, p + 6)

        for s in range(2):
            drain(s)
            extract(s, 510 + s)
            maybe_flush(510 + s)

    out = sc_gather(tt_mu, tt_lv, idx)
    return out[:, :FEAT], out[:, FEAT:]


# speedup vs baseline: 2.8139x; 1.0030x over previous
import jax
import jax.numpy as jnp
from jax import lax
from jax.experimental import pallas as pl
from jax.experimental.pallas import tpu as pltpu
from jax.experimental.pallas import tpu_sc as plsc

BATCH = 16384
FEAT = 64
NC, NS = 2, 16
NW = NC * NS
B_PER_W = BATCH // NW
PG = 128
STAGE_ROWS = 64


def kernel(item_index, mu_table, logvar_table):
    idx = item_index.reshape(BATCH).astype(jnp.int32)
    tt_mu = mu_table.T
    tt_lv = logvar_table.T
    mesh = plsc.VectorSubcoreMesh(core_axis_name="core", subcore_axis_name="subcore")
    out_type = jax.ShapeDtypeStruct((BATCH, 2 * FEAT), jnp.float32)

    @pl.kernel(
        out_type=out_type,
        mesh=mesh,
        scratch_types=[
            pltpu.VMEM((B_PER_W,), jnp.int32),
            pltpu.VMEM((7, 2, FEAT, PG), jnp.float32),
            pltpu.VMEM((STAGE_ROWS, 2 * FEAT), jnp.float32),
            pltpu.SemaphoreType.DMA((7,)),
        ],
        compiler_params=pltpu.CompilerParams(
            use_tc_tiling_on_sc=True, needs_layout_passes=False
        ),
    )
    def sc_gather(mu_hbm, lv_hbm, i_hbm, o_hbm, idx_v, bufs, stage, sems):
        wid = lax.axis_index("subcore") * NC + lax.axis_index("core")
        base = wid * B_PER_W
        pltpu.sync_copy(i_hbm.at[pl.ds(base, B_PER_W)], idx_v)

        iotas = [lax.iota(jnp.int32, 16) + 16 * c for c in range(4)]

        def sload(pos):
            g = (pos >> 4) << 4
            v = idx_v[pl.ds(g, 16)]
            m = lax.iota(jnp.int32, 16) == (pos - g)
            return jnp.sum(jnp.where(m, v, 0))

        def fire(s, pos):
            i = sload(pos)
            po = pl.multiple_of((i >> 7) * PG, PG)
            pltpu.async_copy(mu_hbm.at[:, pl.ds(po, PG)], bufs.at[s, 0], sems.at[s])
            pltpu.async_copy(lv_hbm.at[:, pl.ds(po, PG)], bufs.at[s, 1], sems.at[s])

        def drain(s):
            for _ in range(2):
                pltpu.make_async_copy(
                    mu_hbm.at[:, pl.ds(0, PG)], bufs.at[s, 0], sems.at[s]
                ).wait()

        def extract(s, pos):
            i = sload(pos)
            lanev = jnp.zeros((16,), jnp.int32) + (i & 127)
            slot = pos & (STAGE_ROWS - 1)
            for t in range(2):
                for c in range(4):
                    v = plsc.load_gather(bufs.at[s, t], [iotas[c], lanev])
                    stage[slot, pl.ds(t * FEAT + 16 * c, 16)] = v

        def maybe_flush(last_pos):
            @pl.when((last_pos & (STAGE_ROWS - 1)) == STAGE_ROWS - 1)
            def _():
                pg_row = base + (last_pos >> 6) * STAGE_ROWS
                pltpu.sync_copy(stage, o_hbm.at[pl.ds(pg_row, STAGE_ROWS)])

        for s in range(6):
            fire(s, s)

        @pl.loop(0, 72)
        def _(m):
            p = m * 7
            for s in range(7):
                drain(s)
                fire((s + 6) % 7, p + s + 6)
                extract(s, p + s)
                maybe_flush(p + s)

        for s in range(7):
            p = 504 + s
            drain(s)

            @pl.when(p + 6 < B_PER_W)
            def _():
                fire((s + 6) % 7, p + 6)

            extract(s, p)
            maybe_flush(p)

        drain(0)
        extract(0, 511)
        maybe_flush(511)

    out = sc_gather(tt_mu, tt_lv, idx)
    return out[:, :FEAT], out[:, FEAT:]


# speedup vs baseline: 3.8887x; 1.3820x over previous
import jax
import jax.numpy as jnp
from jax import lax
from jax.experimental import pallas as pl
from jax.experimental.pallas import tpu as pltpu
from jax.experimental.pallas import tpu_sc as plsc

BATCH = 16384
FEAT = 64
NC, NS = 2, 16
NW = NC * NS
B_PER_W = BATCH // NW
PG = 128
STAGE_ROWS = 64
NB = 6
LOOKAHEAD = NB - 2


def kernel(item_index, mu_table, logvar_table):
    idx = item_index.reshape(BATCH).astype(jnp.int32)
    order = jnp.argsort(idx)
    sidx = idx[order]
    inv = jnp.zeros((BATCH,), jnp.int32).at[order].set(lax.iota(jnp.int32, BATCH))

    seg_pages = (sidx >> 7).reshape(NW, B_PER_W)
    newpage = jnp.concatenate(
        [jnp.ones((NW, 1), bool), seg_pages[:, 1:] != seg_pages[:, :-1]], axis=1
    )
    slots = jnp.cumsum(newpage, axis=1).astype(jnp.int32) - 1
    nuniq = slots[:, -1] + 1
    w_rows = jnp.broadcast_to(lax.iota(jnp.int32, NW)[:, None], (NW, B_PER_W))
    upages = jnp.zeros((NW, B_PER_W), jnp.int32).at[w_rows, slots].set(seg_pages)

    tt_mu = mu_table.T
    tt_lv = logvar_table.T
    mesh = plsc.VectorSubcoreMesh(core_axis_name="core", subcore_axis_name="subcore")
    out_type = jax.ShapeDtypeStruct((BATCH, 2 * FEAT), jnp.float32)

    @pl.kernel(
        out_type=out_type,
        mesh=mesh,
        scratch_types=[
            pltpu.VMEM((B_PER_W,), jnp.int32),
            pltpu.VMEM((B_PER_W,), jnp.int32),
            pltpu.VMEM((B_PER_W,), jnp.int32),
            pltpu.VMEM((NW,), jnp.int32),
            pltpu.VMEM((NB, 2, FEAT, PG), jnp.float32),
            pltpu.VMEM((STAGE_ROWS, 2 * FEAT), jnp.float32),
            pltpu.SemaphoreType.DMA((NB,)),
        ],
        compiler_params=pltpu.CompilerParams(
            use_tc_tiling_on_sc=True, needs_layout_passes=False
        ),
    )
    def sc_gather(
        mu_hbm, lv_hbm, si_hbm, sl_hbm, up_hbm, nq_hbm, o_hbm,
        sidx_v, slot_v, upage_v, nq_v, bufs, stage, sems,
    ):
        wid = lax.axis_index("subcore") * NC + lax.axis_index("core")
        base = wid * B_PER_W
        pltpu.sync_copy(si_hbm.at[pl.ds(base, B_PER_W)], sidx_v)
        pltpu.sync_copy(sl_hbm.at[wid], slot_v)
        pltpu.sync_copy(up_hbm.at[wid], upage_v)
        pltpu.sync_copy(nq_hbm, nq_v)

        iotas = [lax.iota(jnp.int32, 16) + 16 * c for c in range(4)]

        def sload(ref, pos):
            g = (pos >> 4) << 4
            v = ref[pl.ds(g, 16)]
            m = lax.iota(jnp.int32, 16) == (pos - g)
            return jnp.sum(jnp.where(m, v, 0))

        nq = sload(nq_v, wid)

        def fire(u):
            b = jnp.remainder(u, NB)
            po = pl.multiple_of(sload(upage_v, u) * PG, PG)
            pltpu.async_copy(mu_hbm.at[:, pl.ds(po, PG)], bufs.at[b, 0], sems.at[b])
            pltpu.async_copy(lv_hbm.at[:, pl.ds(po, PG)], bufs.at[b, 1], sems.at[b])

        def drain(u):
            b = jnp.remainder(u, NB)
            for _ in range(2):
                pltpu.make_async_copy(
                    mu_hbm.at[:, pl.ds(0, PG)], bufs.at[b, 0], sems.at[b]
                ).wait()

        def extract(pos, s):
            b = jnp.remainder(s, NB)
            i = sload(sidx_v, pos)
            lanev = jnp.zeros((16,), jnp.int32) + (i & 127)
            slot = pos & (STAGE_ROWS - 1)
            for t in range(2):
                for c in range(4):
                    v = plsc.load_gather(bufs.at[b, t], [iotas[c], lanev])
                    stage[slot, pl.ds(t * FEAT + 16 * c, 16)] = v

        def body(p, carry):
            fired, drained = carry
            s = sload(slot_v, p)
            fire_end = jnp.minimum(nq, s + LOOKAHEAD + 1)

            @pl.loop(fired, fire_end)
            def _(u):
                fire(u)

            @pl.loop(drained, s + 1)
            def _(u):
                drain(u)

            extract(p, s)

            @pl.when((p & (STAGE_ROWS - 1)) == STAGE_ROWS - 1)
            def _():
                pg_row = base + (p >> 6) * STAGE_ROWS
                pltpu.sync_copy(stage, o_hbm.at[pl.ds(pg_row, STAGE_ROWS)])

            return (jnp.maximum(fired, fire_end), jnp.maximum(drained, s + 1))

        @pl.loop(0, B_PER_W, init_carry=(jnp.int32(0), jnp.int32(0)))
        def _(p, carry):
            return body(p, carry)

    staging = sc_gather(tt_mu, tt_lv, sidx, slots, upages, nuniq)

    inv2 = inv.reshape(NW * 4, 128)

    @pl.kernel(
        out_type=jax.ShapeDtypeStruct((BATCH, 2 * FEAT), jnp.float32),
        mesh=mesh,
        scratch_types=[
            pltpu.VMEM((4, 128), jnp.int32),
            pltpu.VMEM((B_PER_W, 2 * FEAT), jnp.float32),
            pltpu.SemaphoreType.DMA,
        ],
        compiler_params=pltpu.CompilerParams(use_tc_tiling_on_sc=True),
    )
    def sc_unsort(st_hbm, i_hbm, o_hbm, idx_v, rows_v, sem):
        wid = lax.axis_index("subcore") * NC + lax.axis_index("core")
        base = wid * B_PER_W
        pltpu.sync_copy(i_hbm.at[pl.ds(wid * 4, 4)], idx_v)
        copies = [
            pltpu.async_copy(st_hbm.at[idx_v.at[j]], rows_v.at[pl.ds(j * 128, 128)], sem)
            for j in range(4)
        ]
        for c in copies:
            c.wait()
        pltpu.sync_copy(rows_v, o_hbm.at[pl.ds(base, B_PER_W)])

    out = sc_unsort(staging, inv2)
    return out[:, :FEAT], out[:, FEAT:]


# speedup vs baseline: 3.9567x; 1.0175x over previous
import jax
import jax.numpy as jnp
from jax import lax
from jax.experimental import pallas as pl
from jax.experimental.pallas import tpu as pltpu
from jax.experimental.pallas import tpu_sc as plsc

BATCH = 16384
FEAT = 64
NC, NS = 2, 16
NW = NC * NS
B_PER_W = BATCH // NW
PG = 128
STAGE_ROWS = 64
NB = 7
LOOKAHEAD = NB - 2


def kernel(item_index, mu_table, logvar_table):
    idx = item_index.reshape(BATCH).astype(jnp.int32)
    order = jnp.argsort(idx)
    sidx = idx[order]
    inv = jnp.zeros((BATCH,), jnp.int32).at[order].set(lax.iota(jnp.int32, BATCH))

    seg_pages = (sidx >> 7).reshape(NW, B_PER_W)
    newpage = jnp.concatenate(
        [jnp.ones((NW, 1), bool), seg_pages[:, 1:] != seg_pages[:, :-1]], axis=1
    )
    slots = jnp.cumsum(newpage, axis=1).astype(jnp.int32) - 1
    nuniq = slots[:, -1] + 1
    w_rows = jnp.broadcast_to(lax.iota(jnp.int32, NW)[:, None], (NW, B_PER_W))
    upages = jnp.zeros((NW, B_PER_W), jnp.int32).at[w_rows, slots].set(seg_pages)

    tt_mu = mu_table.T
    tt_lv = logvar_table.T
    mesh = plsc.VectorSubcoreMesh(core_axis_name="core", subcore_axis_name="subcore")
    out_type = jax.ShapeDtypeStruct((BATCH, 2 * FEAT), jnp.float32)

    @pl.kernel(
        out_type=out_type,
        mesh=mesh,
        scratch_types=[
            pltpu.VMEM((B_PER_W,), jnp.int32),
            pltpu.VMEM((B_PER_W,), jnp.int32),
            pltpu.VMEM((B_PER_W,), jnp.int32),
            pltpu.VMEM((NW,), jnp.int32),
            pltpu.VMEM((NB, 2, FEAT, PG), jnp.float32),
            pltpu.VMEM((STAGE_ROWS, 2 * FEAT), jnp.float32),
            pltpu.SemaphoreType.DMA((NB,)),
        ],
        compiler_params=pltpu.CompilerParams(
            use_tc_tiling_on_sc=True, needs_layout_passes=False
        ),
    )
    def sc_gather(
        mu_hbm, lv_hbm, si_hbm, sl_hbm, up_hbm, nq_hbm, o_hbm,
        sidx_v, slot_v, upage_v, nq_v, bufs, stage, sems,
    ):
        wid = lax.axis_index("subcore") * NC + lax.axis_index("core")
        base = wid * B_PER_W
        pltpu.sync_copy(si_hbm.at[pl.ds(base, B_PER_W)], sidx_v)
        pltpu.sync_copy(sl_hbm.at[wid], slot_v)
        pltpu.sync_copy(up_hbm.at[wid], upage_v)
        pltpu.sync_copy(nq_hbm, nq_v)

        iotas = [lax.iota(jnp.int32, 16) + 16 * c for c in range(4)]

        def sload(ref, pos):
            g = (pos >> 4) << 4
            v = ref[pl.ds(g, 16)]
            m = lax.iota(jnp.int32, 16) == (pos - g)
            return jnp.sum(jnp.where(m, v, 0))

        nq = sload(nq_v, wid)

        def fire(u):
            b = jnp.remainder(u, NB)
            po = pl.multiple_of(sload(upage_v, u) * PG, PG)
            pltpu.async_copy(mu_hbm.at[:, pl.ds(po, PG)], bufs.at[b, 0], sems.at[b])
            pltpu.async_copy(lv_hbm.at[:, pl.ds(po, PG)], bufs.at[b, 1], sems.at[b])

        def drain(u):
            b = jnp.remainder(u, NB)
            for _ in range(2):
                pltpu.make_async_copy(
                    mu_hbm.at[:, pl.ds(0, PG)], bufs.at[b, 0], sems.at[b]
                ).wait()

        def extract(pos, s):
            b = jnp.remainder(s, NB)
            i = sload(sidx_v, pos)
            lanev = jnp.zeros((16,), jnp.int32) + (i & 127)
            slot = pos & (STAGE_ROWS - 1)
            for t in range(2):
                for c in range(4):
                    v = plsc.load_gather(bufs.at[b, t], [iotas[c], lanev])
                    stage[slot, pl.ds(t * FEAT + 16 * c, 16)] = v

        def body(p, carry):
            fired, drained = carry
            s = sload(slot_v, p)
            fire_end = jnp.minimum(nq, s + LOOKAHEAD + 1)

            @pl.loop(fired, fire_end)
            def _(u):
                fire(u)

            @pl.loop(drained, s + 1)
            def _(u):
                drain(u)

            extract(p, s)

            @pl.when((p & (STAGE_ROWS - 1)) == STAGE_ROWS - 1)
            def _():
                pg_row = base + (p >> 6) * STAGE_ROWS
                pltpu.sync_copy(stage, o_hbm.at[pl.ds(pg_row, STAGE_ROWS)])

            return (jnp.maximum(fired, fire_end), jnp.maximum(drained, s + 1))

        @pl.loop(0, B_PER_W, init_carry=(jnp.int32(0), jnp.int32(0)))
        def _(p, carry):
            return body(p, carry)

    staging = sc_gather(tt_mu, tt_lv, sidx, slots, upages, nuniq)

    inv2 = inv.reshape(NW * 4, 128)

    @pl.kernel(
        out_type=jax.ShapeDtypeStruct((BATCH, 2 * FEAT), jnp.float32),
        mesh=mesh,
        scratch_types=[
            pltpu.VMEM((4, 128), jnp.int32),
            pltpu.VMEM((B_PER_W, 2 * FEAT), jnp.float32),
            pltpu.SemaphoreType.DMA,
        ],
        compiler_params=pltpu.CompilerParams(use_tc_tiling_on_sc=True),
    )
    def sc_unsort(st_hbm, i_hbm, o_hbm, idx_v, rows_v, sem):
        wid = lax.axis_index("subcore") * NC + lax.axis_index("core")
        base = wid * B_PER_W
        pltpu.sync_copy(i_hbm.at[pl.ds(wid * 4, 4)], idx_v)
        copies = [
            pltpu.async_copy(st_hbm.at[idx_v.at[j]], rows_v.at[pl.ds(j * 128, 128)], sem)
            for j in range(4)
        ]
        for c in copies:
            c.wait()
        pltpu.sync_copy(rows_v, o_hbm.at[pl.ds(base, B_PER_W)])

    out = sc_unsort(staging, inv2)
    return out[:, :FEAT], out[:, FEAT:]
